# baseline (speedup 1.0000x reference)
import functools

import jax
import jax.numpy as jnp
from jax import lax
from jax.experimental import pallas as pl
from jax.experimental.pallas import tpu as pltpu
from jax.experimental.pallas import tpu_sc as plsc

NC = 2
NS = 16
D = 8


def _edge_pass(table, src, dst, zeros):
    N, _ = table.shape
    E = src.shape[0]
    NW = NC * NS
    ew = E // NW
    K = 2000
    NB = 4
    nchunk = ew // K
    rpt = N // NS

    mesh = plsc.VectorSubcoreMesh(core_axis_name="c", subcore_axis_name="s")

    @functools.partial(
        pl.kernel,
        out_type=jax.ShapeDtypeStruct((NC, N, D), jnp.float32),
        mesh=mesh,
        scratch_types=(
            [pltpu.VMEM_SHARED((N, D), jnp.float32)]
            + [pltpu.VMEM((K,), jnp.int32) for _ in range(2 * NB)]
            + [pltpu.VMEM((K, D), jnp.float32) for _ in range(NB)]
            + [pltpu.SemaphoreType.DMA for _ in range(2 * NB)]
        ),
        compiler_params=pltpu.CompilerParams(use_tc_tiling_on_sc=False),
    )
    def k(table_h, src_h, dst_h, zeros_h, out_h, acc, *bufs):
        isrc = bufs[0:NB]
        idst = bufs[NB:2 * NB]
        rows = bufs[2 * NB:3 * NB]
        sem_g = bufs[3 * NB:4 * NB]
        sem_s = bufs[4 * NB:5 * NB]
        c = lax.axis_index("c")
        s = lax.axis_index("s")
        wid = s * NC + c
        pltpu.sync_copy(zeros_h, acc.at[pl.ds(s * rpt, rpt)])
        plsc.subcore_barrier()
        base = wid * ew

        def load_idx(i, b):
            off = base + i * K
            pltpu.sync_copy(src_h.at[pl.ds(off, K)], isrc[b])
            pltpu.sync_copy(dst_h.at[pl.ds(off, K)], idst[b])

        def start_gather(b):
            pltpu.async_copy(table_h.at[isrc[b]], rows[b], sem_g[b])

        def wait_gather(b):
            pltpu.make_async_copy(table_h.at[isrc[b]], rows[b],
                                  sem_g[b]).wait()

        def start_scatter(b):
            pltpu.async_copy(rows[b], acc.at[idst[b]], sem_s[b], add=True)

        def wait_scatter(b):
            pltpu.make_async_copy(rows[b], acc.at[idst[b]],
                                  sem_s[b]).wait()

        load_idx(0, 0)
        start_gather(0)
        load_idx(1, 1)
        start_gather(1)

        def body(g, carry):
            for b in range(NB):
                i = g * NB + b
                nb = (b + 2) % NB
                wait_gather(b)
                start_scatter(b)

                @pl.when(i >= 2)
                def _():
                    wait_scatter(nb)

                @pl.when(i + 2 < nchunk)
                def _():
                    load_idx(i + 2, nb)
                    start_gather(nb)
            return carry

        lax.fori_loop(0, nchunk // NB, body, 0)
        for i in range((nchunk // NB) * NB, nchunk):
            b = i % NB
            nb = (i + 2) % NB
            wait_gather(b)
            start_scatter(b)
            if i >= 2:
                wait_scatter(nb)
            if i + 2 < nchunk:
                load_idx(i + 2, nb)
                start_gather(nb)
        wait_scatter((nchunk - 2) % NB)
        wait_scatter((nchunk - 1) % NB)
        plsc.subcore_barrier()
        pltpu.sync_copy(acc.at[pl.ds(s * rpt, rpt)],
                        out_h.at[c, pl.ds(s * rpt, rpt)])

    return k(table, src, dst, zeros)


def _dense1(pT, xT, Wl1, Wr1, b1):
    N = xT.shape[1]

    def body(p_ref, x_ref, wl1, wr1, b1r, h_ref, ci_ref):
        cnt = p_ref[0, 3, :] + p_ref[1, 3, :]
        ci = 1.0 / jnp.maximum(cnt, 1.0)
        ci_ref[...] = ci
        a = [(p_ref[0, kk, :] + p_ref[1, kk, :]) * ci for kk in range(3)]
        xs = [x_ref[kk, :] for kk in range(3)]
        for j in range(8):
            v = a[0] * wl1[j, 0]
            for kk in range(1, 3):
                v = v + a[kk] * wl1[j, kk]
            for kk in range(3):
                v = v + xs[kk] * wr1[j, kk]
            h_ref[j, :] = jnp.maximum(v + b1r[j], 0.0)

    smem = pl.BlockSpec(memory_space=pltpu.SMEM)
    vmem = pl.BlockSpec(memory_space=pltpu.VMEM)
    return pl.pallas_call(
        body,
        out_shape=[
            jax.ShapeDtypeStruct((8, N), jnp.float32),
            jax.ShapeDtypeStruct((N,), jnp.float32),
        ],
        in_specs=[vmem, vmem, smem, smem, smem],
        out_specs=[vmem, vmem],
    )(pT, xT, Wl1, Wr1, b1)


def _dense2(p2T, hT, cinv, Wl2, Wr2, b2):
    N = cinv.shape[0]

    def body(p_ref, h_ref, ci_ref, wl2, wr2, b2r, o_ref):
        ci = ci_ref[...]
        a = [(p_ref[0, j, :] + p_ref[1, j, :]) * ci for j in range(8)]
        hs = [h_ref[j, :] for j in range(8)]
        for cc in range(2):
            v = a[0] * wl2[cc, 0]
            for j in range(1, 8):
                v = v + a[j] * wl2[cc, j]
            for j in range(8):
                v = v + hs[j] * wr2[cc, j]
            o_ref[cc, :] = v + b2r[cc]

    smem = pl.BlockSpec(memory_space=pltpu.SMEM)
    vmem = pl.BlockSpec(memory_space=pltpu.VMEM)
    return pl.pallas_call(
        body,
        out_shape=jax.ShapeDtypeStruct((2, N), jnp.float32),
        in_specs=[vmem, vmem, vmem, smem, smem, smem],
        out_specs=vmem,
    )(p2T, hT, cinv, Wl2, Wr2, b2)


def kernel(x, edge_index, Wl1, Wr1, b1, Wl2, Wr2, b2):
    N = x.shape[0]
    src = edge_index[0]
    dst = edge_index[1]
    xa = jnp.concatenate(
        [x, jnp.ones((N, 1), x.dtype), jnp.zeros((N, 4), x.dtype)], axis=1)
    z8 = jnp.zeros((N // NS, D), jnp.float32)
    p1 = _edge_pass(xa, src, dst, z8)
    hT, cinv = _dense1(p1.transpose(0, 2, 1), x.T, Wl1, Wr1, b1)
    h = hT.T
    p2 = _edge_pass(h, src, dst, z8)
    outT = _dense2(p2.transpose(0, 2, 1), hT, cinv, Wl2, Wr2, b2)
    return outT.T


# speedup vs baseline: 1.2199x; 1.2199x over previous
import functools

import jax
import jax.numpy as jnp
from jax import lax
from jax.experimental import pallas as pl
from jax.experimental.pallas import tpu as pltpu
from jax.experimental.pallas import tpu_sc as plsc

NC = 2
NS = 16
D = 8


def _edge_pass(table, src, dst, zeros):
    N, _ = table.shape
    E = src.shape[0]
    NW = NC * NS
    ew = E // NW
    K = 2000
    NB = 4
    nchunk = ew // K
    rpt = N // NS

    mesh = plsc.VectorSubcoreMesh(core_axis_name="c", subcore_axis_name="s")

    @functools.partial(
        pl.kernel,
        out_type=jax.ShapeDtypeStruct((NC, N, D), jnp.float32),
        mesh=mesh,
        scratch_types=(
            [pltpu.VMEM_SHARED((N, D), jnp.float32)]
            + [pltpu.VMEM((K,), jnp.int32) for _ in range(2 * NB)]
            + [pltpu.VMEM((K, D), jnp.float32) for _ in range(NB)]
            + [pltpu.SemaphoreType.DMA for _ in range(2 * NB)]
        ),
        compiler_params=pltpu.CompilerParams(use_tc_tiling_on_sc=False),
    )
    def k(table_h, src_h, dst_h, zeros_h, out_h, acc, *bufs):
        isrc = bufs[0:NB]
        idst = bufs[NB:2 * NB]
        rows = bufs[2 * NB:3 * NB]
        sem_g = bufs[3 * NB:4 * NB]
        sem_s = bufs[4 * NB:5 * NB]
        c = lax.axis_index("c")
        s = lax.axis_index("s")
        wid = s * NC + c
        pltpu.sync_copy(zeros_h, acc.at[pl.ds(s * rpt, rpt)])
        plsc.subcore_barrier()
        base = wid * ew

        def load_idx(i, b):
            off = base + i * K
            pltpu.sync_copy(src_h.at[pl.ds(off, K)], isrc[b])
            pltpu.sync_copy(dst_h.at[pl.ds(off, K)], idst[b])

        def start_gather(b):
            pltpu.async_copy(table_h.at[isrc[b]], rows[b], sem_g[b])

        def wait_gather(b):
            pltpu.make_async_copy(table_h.at[isrc[b]], rows[b],
                                  sem_g[b]).wait()

        def start_scatter(b):
            pltpu.async_copy(rows[b], acc.at[idst[b]], sem_s[b], add=True)

        def wait_scatter(b):
            pltpu.make_async_copy(rows[b], acc.at[idst[b]],
                                  sem_s[b]).wait()

        load_idx(0, 0)
        start_gather(0)
        load_idx(1, 1)
        start_gather(1)

        def body(g, carry):
            for b in range(NB):
                i = g * NB + b
                nb = (b + 2) % NB
                wait_gather(b)
                start_scatter(b)

                @pl.when(i >= 2)
                def _():
                    wait_scatter(nb)

                @pl.when(i + 2 < nchunk)
                def _():
                    load_idx(i + 2, nb)
                    start_gather(nb)
            return carry

        lax.fori_loop(0, nchunk // NB, body, 0)
        for i in range((nchunk // NB) * NB, nchunk):
            b = i % NB
            nb = (i + 2) % NB
            wait_gather(b)
            start_scatter(b)
            if i >= 2:
                wait_scatter(nb)
            if i + 2 < nchunk:
                load_idx(i + 2, nb)
                start_gather(nb)
        wait_scatter((nchunk - 2) % NB)
        wait_scatter((nchunk - 1) % NB)
        plsc.subcore_barrier()
        pltpu.sync_copy(acc.at[pl.ds(s * rpt, rpt)],
                        out_h.at[c, pl.ds(s * rpt, rpt)])

    return k(table, src, dst, zeros)


def _dense1(p1b, xab, M1, M2, P):
    Nr = xab.shape[0]

    def body(p_ref, xa_ref, m1, m2, pm, h_ref, ci_ref):
        s = p_ref[0] + p_ref[1]
        cnt_b = jnp.dot(s, pm[...], preferred_element_type=jnp.float32)
        ci_b = 1.0 / jnp.maximum(cnt_b, 1.0)
        ci_ref[...] = ci_b
        a = s * ci_b
        h_ref[...] = jnp.maximum(
            jnp.dot(a, m1[...], preferred_element_type=jnp.float32)
            + jnp.dot(xa_ref[...], m2[...],
                      preferred_element_type=jnp.float32), 0.0)

    vmem = pl.BlockSpec(memory_space=pltpu.VMEM)
    return pl.pallas_call(
        body,
        out_shape=[
            jax.ShapeDtypeStruct((Nr, 128), jnp.float32),
            jax.ShapeDtypeStruct((Nr, 128), jnp.float32),
        ],
        in_specs=[vmem] * 5,
        out_specs=[vmem, vmem],
    )(p1b, xab, M1, M2, P)


def _dense2(p2b, hb, cib, M3, M4, b2v):
    Nr = hb.shape[0]

    def body(p_ref, h_ref, ci_ref, m3, m4, b2r, o_ref):
        a2 = (p_ref[0] + p_ref[1]) * ci_ref[...]
        o_ref[...] = (
            jnp.dot(a2, m3[...], preferred_element_type=jnp.float32)
            + jnp.dot(h_ref[...], m4[...],
                      preferred_element_type=jnp.float32)
            + b2r[...])

    vmem = pl.BlockSpec(memory_space=pltpu.VMEM)
    return pl.pallas_call(
        body,
        out_shape=jax.ShapeDtypeStruct((Nr, 128), jnp.float32),
        in_specs=[vmem] * 6,
        out_specs=vmem,
    )(p2b, hb, cib, M3, M4, b2v)


def kernel(x, edge_index, Wl1, Wr1, b1, Wl2, Wr2, b2):
    N = x.shape[0]
    G = 16
    Nr = N // G
    src = edge_index[0]
    dst = edge_index[1]
    xa = jnp.concatenate(
        [x, jnp.ones((N, 1), x.dtype), jnp.zeros((N, 4), x.dtype)], axis=1)
    z8 = jnp.zeros((N // NS, D), jnp.float32)

    eye = jnp.eye(G, dtype=jnp.float32)
    B1 = jnp.zeros((8, 8), jnp.float32).at[:3, :].set(Wl1.T)
    B2 = jnp.zeros((8, 8), jnp.float32).at[:3, :].set(Wr1.T).at[3, :].set(b1)
    BP = jnp.zeros((8, 8), jnp.float32).at[3, :].set(1.0)
    B3 = jnp.zeros((8, 8), jnp.float32).at[:, :2].set(Wl2.T)
    B4 = jnp.zeros((8, 8), jnp.float32).at[:, :2].set(Wr2.T)
    M1 = jnp.kron(eye, B1)
    M2 = jnp.kron(eye, B2)
    P = jnp.kron(eye, BP)
    M3 = jnp.kron(eye, B3)
    M4 = jnp.kron(eye, B4)
    b2v = jnp.tile(jnp.concatenate([b2, jnp.zeros((6,), jnp.float32)]), G)

    p1 = _edge_pass(xa, src, dst, z8)
    hb, cib = _dense1(p1.reshape(NC, Nr, 128), xa.reshape(Nr, 128),
                      M1, M2, P)
    h = hb.reshape(N, 8)
    p2 = _edge_pass(h, src, dst, z8)
    outb = _dense2(p2.reshape(NC, Nr, 128), hb, cib, M3, M4, b2v)
    return outb.reshape(N, 8)[:, :2]
